# baseline (speedup 1.0000x reference)
import functools
from typing import Any

import jax
import jax.numpy as jnp
import numpy as np
from jax.experimental import pallas as pl
from jax.experimental.pallas import tpu as pltpu

V = 50304
D = 1024
L = 4
H = 16
KV = 8
DH = 64
E = 8
FF = 512
K = 2
S = 2048
CAP = 512
EPS_RMS = 1e-6
EPS_QK = 1e-5

HP = jax.lax.Precision.HIGHEST

BF = jnp.bfloat16

BM = 256
BQ = 256
BK = 256
BV = 384
NBV = V // BV


def _dot(a, b, precision=None):
    return jax.lax.dot_general(a.astype(BF), b.astype(BF), (((1,), (0,)), ((), ())),
                               preferred_element_type=jnp.float32,
                               precision=precision)


def _dot_t(a, b, precision=None):
    return jax.lax.dot_general(a.astype(BF), b.astype(BF), (((1,), (1,)), ((), ())),
                               preferred_element_type=jnp.float32,
                               precision=precision)


def _gather_body(ids_ref, emb_ref, o_ref):
    o_ref[...] = emb_ref[...]


def _embed_gather(ids, embed):
    out = pl.pallas_call(
        _gather_body,
        grid_spec=pltpu.PrefetchScalarGridSpec(
            num_scalar_prefetch=1,
            grid=(S,),
            in_specs=[pl.BlockSpec((1, 1, D), lambda i, ids: (ids[0, i], 0, 0))],
            out_specs=pl.BlockSpec((1, 1, D), lambda i, ids: (i, 0, 0)),
        ),
        out_shape=jax.ShapeDtypeStruct((S, 1, D), jnp.float32),
    )(ids, embed.reshape(V, 1, D))
    return out.reshape(S, D)


def _proj_body(x_ref, nw_ref, w_ref, o_ref):
    x = x_ref[...]
    ms = jnp.mean(x * x, axis=1, keepdims=True)
    hn = x * jax.lax.rsqrt(ms + EPS_RMS) * nw_ref[...]
    o_ref[...] = _dot(hn, w_ref[...])


def _qkvg_proj(x, nw, wcat):
    n_out = wcat.shape[1]
    return pl.pallas_call(
        _proj_body,
        grid=(S // BM,),
        in_specs=[
            pl.BlockSpec((BM, D), lambda i: (i, 0)),
            pl.BlockSpec((1, D), lambda i: (0, 0)),
            pl.BlockSpec((D, n_out), lambda i: (0, 0)),
        ],
        out_specs=pl.BlockSpec((BM, n_out), lambda i: (i, 0)),
        out_shape=jax.ShapeDtypeStruct((S, n_out), jnp.float32),
    )(x, nw.reshape(1, D), wcat)


def _qkprep_body(z_ref, qn_ref, kn_ref, cos_ref, sin_ref, o_ref):
    c = pl.program_id(0)
    x = z_ref[...]
    nw = jnp.where(c < H // 2, qn_ref[...], kn_ref[...])
    cs = cos_ref[...]
    sn = sin_ref[...]
    halves = []
    for t in range(2):
        xh = x[:, t * DH:(t + 1) * DH]
        ms = jnp.mean(xh * xh, axis=1, keepdims=True)
        xh = xh * jax.lax.rsqrt(ms + EPS_QK) * nw
        x1 = xh[:, : DH // 2]
        x2 = xh[:, DH // 2:]
        rot = jnp.concatenate([-x2, x1], axis=1)
        halves.append(xh * cs + rot * sn)
    o_ref[...] = jnp.concatenate(halves, axis=1)


def _qk_prep(z_qk, qn, kn, cos, sin):
    return pl.pallas_call(
        _qkprep_body,
        grid=((H + KV) // 2,),
        in_specs=[
            pl.BlockSpec((S, 2 * DH), lambda c: (0, c)),
            pl.BlockSpec((1, DH), lambda c: (0, 0)),
            pl.BlockSpec((1, DH), lambda c: (0, 0)),
            pl.BlockSpec((S, DH), lambda c: (0, 0)),
            pl.BlockSpec((S, DH), lambda c: (0, 0)),
        ],
        out_specs=pl.BlockSpec((S, 2 * DH), lambda c: (0, c)),
        out_shape=jax.ShapeDtypeStruct((S, (H + KV) * DH), jnp.float32),
    )(z_qk, qn.reshape(1, DH), kn.reshape(1, DH), cos, sin)


def _attn_body(q_ref, k_ref, v_ref, g_ref, o_ref, p_ref):
    j = pl.program_id(1)
    q = q_ref[...]
    scale = 1.0 / float(np.sqrt(DH))
    row = j * BQ + jax.lax.broadcasted_iota(jnp.int32, (BQ, BK), 0)
    nchunk = S // BK
    outs = []
    for h in range(4):
        qh = q[:, h * DH:(h + 1) * DH]
        kvh = h // 2

        def score_step(kk, carry):
            base = pl.multiple_of(kk * BK, BK)
            kh = k_ref[pl.ds(base, BK), kvh * DH:(kvh + 1) * DH]
            s = _dot_t(qh, kh) * scale
            col = kk * BK + jax.lax.broadcasted_iota(jnp.int32, (BQ, BK), 1)
            s = jnp.where(col <= row, s, -1e9)
            p_ref[h * BQ:(h + 1) * BQ, pl.ds(base, BK)] = s
            return carry

        jax.lax.fori_loop(0, j + 1, score_step, 0)

        def fill_step(kk, carry):
            base = pl.multiple_of(kk * BK, BK)
            p_ref[h * BQ:(h + 1) * BQ, pl.ds(base, BK)] = jnp.full(
                (BQ, BK), -1e9, jnp.float32)
            return carry

        jax.lax.fori_loop(j + 1, nchunk, fill_step, 0)

        sfull = p_ref[h * BQ:(h + 1) * BQ, :]
        m = jnp.max(sfull, axis=1, keepdims=True)
        p = jnp.exp(sfull - m)
        den = jnp.sum(p, axis=1, keepdims=True)
        a = p / den
        vh = v_ref[:, kvh * DH:(kvh + 1) * DH]
        outs.append(_dot(a, vh))
    o = jnp.concatenate(outs, axis=1)
    gate = jax.nn.sigmoid(g_ref[...])
    o_ref[...] = o * gate


def _attn_core(qk, z):
    return pl.pallas_call(
        _attn_body,
        grid=(H // 4, S // BQ),
        in_specs=[
            pl.BlockSpec((BQ, 4 * DH), lambda g, j: (j, g)),
            pl.BlockSpec((S, 2 * DH), lambda g, j: (0, 8 + g)),
            pl.BlockSpec((S, 2 * DH), lambda g, j: (0, 12 + g)),
            pl.BlockSpec((BQ, 4 * DH), lambda g, j: (j, 8 + g)),
        ],
        out_specs=pl.BlockSpec((BQ, 4 * DH), lambda g, j: (j, g)),
        out_shape=jax.ShapeDtypeStruct((S, H * DH), jnp.float32),
        scratch_shapes=[pltpu.VMEM((4 * BQ, S), jnp.float32)],
    )(qk, qk, z, z)


def _outproj_body(o_ref, w_ref, x_ref, y_ref):
    y_ref[...] = x_ref[...] + _dot(o_ref[...], w_ref[...])


def _out_proj(o, wo, x):
    return pl.pallas_call(
        _outproj_body,
        grid=(S // BM,),
        in_specs=[
            pl.BlockSpec((BM, H * DH), lambda i: (i, 0)),
            pl.BlockSpec((H * DH, D), lambda i: (0, 0)),
            pl.BlockSpec((BM, D), lambda i: (i, 0)),
        ],
        out_specs=pl.BlockSpec((BM, D), lambda i: (i, 0)),
        out_shape=jax.ShapeDtypeStruct((S, D), jnp.float32),
    )(o, wo, x)


def _route_body(x_ref, nw_ref, wr_ref, hn_ref, widx_ref, gidx_ref,
                wslot_ref, aux_ref):
    x = x_ref[...]
    ms = jnp.mean(x * x, axis=1, keepdims=True)
    hn = x * jax.lax.rsqrt(ms + EPS_RMS) * nw_ref[...]
    hn_ref[...] = hn
    rl = jax.lax.dot_general(wr_ref[...].astype(BF), hn.astype(BF),
                             (((0,), (1,)), ((), ())),
                             preferred_element_type=jnp.float32)
    mx = jnp.max(rl, axis=0, keepdims=True)
    ex = jnp.exp(rl - mx)
    probs = ex / jnp.sum(ex, axis=0, keepdims=True)
    iota_e = jax.lax.broadcasted_iota(jnp.int32, (E, S), 0)
    m1 = jnp.max(probs, axis=0, keepdims=True)
    a1 = jnp.min(jnp.where(probs >= m1, iota_e, E), axis=0, keepdims=True)
    probs2 = jnp.where(iota_e == a1, -1.0, probs)
    m2 = jnp.max(probs2, axis=0, keepdims=True)
    a2 = jnp.min(jnp.where(probs2 >= m2, iota_e, E), axis=0, keepdims=True)
    den = m1 + m2
    oh1 = (iota_e == a1).astype(jnp.float32)
    oh2 = (iota_e == a2).astype(jnp.float32)
    mm = jnp.concatenate([oh1, oh2], axis=1)
    iu = jax.lax.broadcasted_iota(jnp.int32, (128, 128), 0)
    ju = jax.lax.broadcasted_iota(jnp.int32, (128, 128), 1)
    triu = (iu <= ju).astype(jnp.float32)
    base = jnp.zeros((E, 1), jnp.float32)
    pos_chunks = []
    for c in range(2 * S // 128):
        seg = mm[:, c * 128:(c + 1) * 128]
        cs = _dot(seg, triu)
        pos_chunks.append(cs - 1.0 + base)
        base = base + cs[:, -1:]
    pos = jnp.concatenate(pos_chunks, axis=1)
    p = jnp.sum(pos * mm, axis=0, keepdims=True)
    keep = p < float(CAP)
    eid = jnp.concatenate([a1, a2], axis=1)
    pc = jnp.clip(p, 0.0, float(CAP - 1)).astype(jnp.int32)
    idx = eid * CAP + pc
    widx_ref[...] = jnp.where(keep, idx, E * CAP)
    gidx_ref[...] = jnp.where(keep, idx, 0)
    gv = jnp.concatenate([m1 / den, m2 / den], axis=1)
    wslot_ref[...] = jnp.where(keep, gv, 0.0)
    f = jnp.mean(oh1, axis=1, keepdims=True)
    pmean = jnp.mean(probs, axis=1, keepdims=True)
    aux_ref[...] = float(E) * jnp.sum(f * pmean, axis=0, keepdims=True)


def _route(x, nw, wr):
    return pl.pallas_call(
        _route_body,
        grid=(1,),
        in_specs=[
            pl.BlockSpec((S, D), lambda i: (0, 0)),
            pl.BlockSpec((1, D), lambda i: (0, 0)),
            pl.BlockSpec((D, E), lambda i: (0, 0)),
        ],
        out_specs=[
            pl.BlockSpec((S, D), lambda i: (0, 0)),
            pl.BlockSpec((1, 2 * S), lambda i: (0, 0)),
            pl.BlockSpec((1, 2 * S), lambda i: (0, 0)),
            pl.BlockSpec((1, 2 * S), lambda i: (0, 0)),
            pl.BlockSpec((1, 1), lambda i: (0, 0)),
        ],
        out_shape=[
            jax.ShapeDtypeStruct((S, D), jnp.float32),
            jax.ShapeDtypeStruct((1, 2 * S), jnp.int32),
            jax.ShapeDtypeStruct((1, 2 * S), jnp.int32),
            jax.ShapeDtypeStruct((1, 2 * S), jnp.float32),
            jax.ShapeDtypeStruct((1, 1), jnp.float32),
        ],
    )(x, nw.reshape(1, D), wr)


def _dispatch_body(widx_ref, zb_ref, hn_ref, buf_ref):
    del widx_ref, zb_ref
    buf_ref[...] = hn_ref[...]


def _dispatch(widx, hn2, zbuf):
    out = pl.pallas_call(
        _dispatch_body,
        grid_spec=pltpu.PrefetchScalarGridSpec(
            num_scalar_prefetch=1,
            grid=(K, S),
            in_specs=[
                pl.BlockSpec((1, 1, D), lambda k, i, w: (0, 0, 0)),
                pl.BlockSpec((1, 1, D), lambda k, i, w: (i, 0, 0)),
            ],
            out_specs=pl.BlockSpec((1, 1, D),
                                   lambda k, i, w: (w[0, k * S + i], 0, 0)),
        ),
        out_shape=jax.ShapeDtypeStruct((E * CAP + 1, 1, D), jnp.float32),
        input_output_aliases={1: 0},
    )(widx, zbuf.reshape(E * CAP + 1, 1, D), hn2.reshape(S, 1, D))
    return out.reshape(E * CAP + 1, D)


def _expert_body(b_ref, w1_ref, w3_ref, w2_ref, o_ref):
    b = b_ref[...]
    h1 = _dot(b, w1_ref[0])
    h3 = _dot(b, w3_ref[0])
    hh = h1 * jax.nn.sigmoid(h1) * h3
    o_ref[...] = _dot(hh, w2_ref[0])


def _expert_ffn(buf, w1, w3, w2):
    return pl.pallas_call(
        _expert_body,
        grid=(E,),
        in_specs=[
            pl.BlockSpec((CAP, D), lambda e: (e, 0)),
            pl.BlockSpec((1, D, FF), lambda e: (e, 0, 0)),
            pl.BlockSpec((1, D, FF), lambda e: (e, 0, 0)),
            pl.BlockSpec((1, FF, D), lambda e: (e, 0, 0)),
        ],
        out_specs=pl.BlockSpec((CAP, D), lambda e: (e, 0)),
        out_shape=jax.ShapeDtypeStruct((E * CAP, D), jnp.float32),
    )(buf, w1, w3, w2)


def _combine_body(gidx_ref, ws_ref, e0_ref, e1_ref, x_ref, y_ref):
    del gidx_ref
    i = pl.program_id(0)
    w0 = ws_ref[0, i]
    w1 = ws_ref[0, S + i]
    y_ref[...] = x_ref[...] + e0_ref[...] * w0 + e1_ref[...] * w1


def _combine(gidx, wslot, eo, x):
    eo3 = eo.reshape(E * CAP, 1, D)
    out = pl.pallas_call(
        _combine_body,
        grid_spec=pltpu.PrefetchScalarGridSpec(
            num_scalar_prefetch=2,
            grid=(S,),
            in_specs=[
                pl.BlockSpec((1, 1, D), lambda i, g, w: (g[0, i], 0, 0)),
                pl.BlockSpec((1, 1, D), lambda i, g, w: (g[0, S + i], 0, 0)),
                pl.BlockSpec((1, 1, D), lambda i, g, w: (i, 0, 0)),
            ],
            out_specs=pl.BlockSpec((1, 1, D), lambda i, g, w: (i, 0, 0)),
        ),
        out_shape=jax.ShapeDtypeStruct((S, 1, D), jnp.float32),
    )(gidx, wslot, eo3, eo3, x.reshape(S, 1, D))
    return out.reshape(S, D)


def _fnorm_body(x_ref, nw_ref, o_ref):
    x = x_ref[...]
    ms = jnp.mean(x * x, axis=1, keepdims=True)
    o_ref[...] = x * jax.lax.rsqrt(ms + EPS_RMS) * nw_ref[...]


def _final_norm(x, nw):
    return pl.pallas_call(
        _fnorm_body,
        grid=(S // BM,),
        in_specs=[
            pl.BlockSpec((BM, D), lambda i: (i, 0)),
            pl.BlockSpec((1, D), lambda i: (0, 0)),
        ],
        out_specs=pl.BlockSpec((BM, D), lambda i: (i, 0)),
        out_shape=jax.ShapeDtypeStruct((S, D), jnp.float32),
    )(x, nw.reshape(1, D))


def _logits_body(xn_ref, emb_ref, o_ref):
    xb = xn_ref[...].astype(jnp.bfloat16)
    eb = emb_ref[...].astype(jnp.bfloat16)
    o_ref[...] = jax.lax.dot_general(xb, eb, (((1,), (1,)), ((), ())),
                                     preferred_element_type=jnp.float32)


def _logits(xn, embed):
    return pl.pallas_call(
        _logits_body,
        grid=(NBV,),
        in_specs=[
            pl.BlockSpec((S, D), lambda j: (0, 0)),
            pl.BlockSpec((BV, D), lambda j: (j, 0)),
        ],
        out_specs=pl.BlockSpec((S, BV), lambda j: (0, j)),
        out_shape=jax.ShapeDtypeStruct((S, V), jnp.float32),
    )(xn, embed)


def _rope_tables():
    inv = 1.0 / (10000.0 ** (jnp.arange(0, DH, 2, dtype=jnp.float32) / DH))
    t = jnp.arange(S, dtype=jnp.float32)
    fr = jnp.outer(t, inv)
    emb = jnp.concatenate([fr, fr], axis=-1)
    return jnp.cos(emb), jnp.sin(emb)


def kernel(input_ids, embed, attn_norm_w, ffn_norm_w, ln_f_w, q_norm_w,
           k_norm_w, Wq, Wk, Wv, Wo, Wg, Wr, W1, W3, W2):
    ids = input_ids.reshape(1, S).astype(jnp.int32)
    cos, sin = _rope_tables()
    x = _embed_gather(ids, embed)
    total_aux = jnp.zeros((), jnp.float32)
    zbuf = jnp.zeros((E * CAP + 1, D), jnp.float32)
    for l in range(L):
        wcat = jnp.concatenate([Wq[l], Wk[l], Wv[l], Wg[l]], axis=1)
        z = _qkvg_proj(x, attn_norm_w[l], wcat)
        qk = _qk_prep(z, q_norm_w[l], k_norm_w[l], cos, sin)
        o = _attn_core(qk, z)
        x = _out_proj(o, Wo[l], x)
        hn2, widx, gidx, wslot, aux = _route(x, ffn_norm_w[l], Wr[l])
        buf = _dispatch(widx, hn2, zbuf)
        eo = _expert_ffn(buf, W1[l], W3[l], W2[l])
        x = _combine(gidx, wslot, eo, x)
        total_aux = total_aux + aux[0, 0]
    xn = _final_norm(x, ln_f_w)
    logits = _logits(xn, embed)
    return logits.reshape(1, S, V), total_aux
